# baseline (speedup 1.0000x reference)
import functools

import jax
import jax.numpy as jnp
from jax import lax
from jax.experimental import pallas as pl
from jax.experimental.pallas import tpu as pltpu
from jax.experimental.pallas import tpu_sc as plsc

_A = 4
_N = 1024
_C = 16
_K = 8192
_D = _A * _C

_IB = 128
_NBLK = _N // _IB

_NC = 2
_NS = 16
_NW = _NC * _NS

_GB_CH = 2
_GB_W = 128


def _diag_kernel(t_hbm, d_ref, vbuf, sems):
    copies = []
    for ib in range(_NBLK):
        cp = pltpu.make_async_copy(
            t_hbm.at[0, pl.ds(ib * 64, 64)],
            vbuf.at[ib],
            sems.at[ib])
        cp.start()
        copies.append(cp)
    m = (lax.broadcasted_iota(jnp.int32, (_IB, 1, _IB), 0) ==
         lax.broadcasted_iota(jnp.int32, (_IB, 1, _IB), 2)).astype(jnp.float32)
    del m
    for ib in range(_NBLK):
        copies[ib].wait()
        s = jnp.sum(vbuf[ib])
        d_ref[pl.ds(ib * _IB, _IB), :] = jnp.full((_IB, _D), s, jnp.float32)


def _diag_extract(t):
    return pl.pallas_call(
        _diag_kernel,
        in_specs=[pl.BlockSpec(memory_space=pltpu.MemorySpace.HBM)],
        out_specs=pl.BlockSpec(memory_space=pltpu.MemorySpace.VMEM),
        out_shape=jax.ShapeDtypeStruct((_N, _D), jnp.float32),
        scratch_shapes=[
            pltpu.VMEM((_NBLK, 64, _C, _N), jnp.float32),
            pltpu.SemaphoreType.DMA((_NBLK,)),
        ],
    )(t)


@functools.partial(
    pl.kernel,
    out_type=jax.ShapeDtypeStruct((_K, _D), jnp.float32),
    mesh=plsc.VectorSubcoreMesh(core_axis_name="c", subcore_axis_name="s"),
    compiler_params=pltpu.CompilerParams(use_tc_tiling_on_sc=False),
    scratch_types=[
        pltpu.VMEM((_GB_CH, _GB_W), jnp.int32),
        pltpu.VMEM((_GB_CH, _GB_W, _D), jnp.float32),
        pltpu.SemaphoreType.DMA,
    ],
)
def _row_gather(d_hbm, idx_hbm, out_hbm, idx_v, rows_v, sem):
    wid = lax.axis_index("s") * _NC + lax.axis_index("c")
    base = wid * _GB_CH * _GB_W
    for j in range(_GB_CH):
        pltpu.sync_copy(idx_hbm.at[pl.ds(base + j * _GB_W, _GB_W)],
                        idx_v.at[j])
    copies = [
        pltpu.async_copy(d_hbm.at[idx_v.at[j]], rows_v.at[j], sem)
        for j in range(_GB_CH)
    ]
    for cp in copies:
        cp.wait()
    for j in range(_GB_CH):
        pltpu.sync_copy(rows_v.at[j],
                        out_hbm.at[pl.ds(base + j * _GB_W, _GB_W)])


def kernel(t, idx):
    d = _diag_extract(t)
    out = _row_gather(d, idx.astype(jnp.int32))
    return out.reshape(_K, _A, _C)


# speedup vs baseline: 1.0230x; 1.0230x over previous
import functools

import jax
import jax.numpy as jnp
from jax import lax
from jax.experimental import pallas as pl
from jax.experimental.pallas import tpu as pltpu
from jax.experimental.pallas import tpu_sc as plsc

_A = 4
_N = 1024
_C = 16
_K = 8192
_D = _A * _C

_IB = 128
_NBLK = _N // _IB

_NC = 2
_NS = 16
_NW = _NC * _NS

_GB_CH = 2
_GB_W = 128


def _diag_kernel(t_hbm, d_ref, vbuf, sems):
    copies = []
    for ib in range(_NBLK):
        cp = pltpu.make_async_copy(
            t_hbm.at[:, pl.ds(ib * _IB, _IB), :, pl.ds(ib * _IB, _IB)],
            vbuf.at[ib],
            sems.at[ib])
        cp.start()
        copies.append(cp)
    m = (lax.broadcasted_iota(jnp.int32, (_IB, 1, _IB), 0) ==
         lax.broadcasted_iota(jnp.int32, (_IB, 1, _IB), 2)).astype(jnp.float32)
    for ib in range(_NBLK):
        copies[ib].wait()
        parts = []
        for a in range(_A):
            parts.append(jnp.sum(vbuf[ib, a] * m, axis=-1))
        d_ref[pl.ds(ib * _IB, _IB), :] = jnp.concatenate(parts, axis=-1)


def _diag_extract(t):
    return pl.pallas_call(
        _diag_kernel,
        in_specs=[pl.BlockSpec(memory_space=pltpu.MemorySpace.HBM)],
        out_specs=pl.BlockSpec(memory_space=pltpu.MemorySpace.VMEM),
        out_shape=jax.ShapeDtypeStruct((_N, _D), jnp.float32),
        scratch_shapes=[
            pltpu.VMEM((_NBLK, _A, _IB, _C, _IB), jnp.float32),
            pltpu.SemaphoreType.DMA((_NBLK,)),
        ],
    )(t)


@functools.partial(
    pl.kernel,
    out_type=jax.ShapeDtypeStruct((_K, _D), jnp.float32),
    mesh=plsc.VectorSubcoreMesh(core_axis_name="c", subcore_axis_name="s"),
    compiler_params=pltpu.CompilerParams(use_tc_tiling_on_sc=False),
    scratch_types=[
        pltpu.VMEM((_GB_CH, _GB_W), jnp.int32),
        pltpu.VMEM((_GB_CH, _GB_W, _D), jnp.float32),
        pltpu.SemaphoreType.DMA,
    ],
)
def _row_gather(d_hbm, idx_hbm, out_hbm, idx_v, rows_v, sem):
    wid = lax.axis_index("s") * _NC + lax.axis_index("c")
    base = wid * _GB_CH * _GB_W
    for j in range(_GB_CH):
        pltpu.sync_copy(idx_hbm.at[pl.ds(base + j * _GB_W, _GB_W)],
                        idx_v.at[j])
    copies = [
        pltpu.async_copy(d_hbm.at[idx_v.at[j]], rows_v.at[j], sem)
        for j in range(_GB_CH)
    ]
    for cp in copies:
        cp.wait()
    for j in range(_GB_CH):
        pltpu.sync_copy(rows_v.at[j],
                        out_hbm.at[pl.ds(base + j * _GB_W, _GB_W)])


def kernel(t, idx):
    d = _diag_extract(t)
    out = _row_gather(d, idx.astype(jnp.int32))
    return out.reshape(_K, _A, _C)
